# baseline (speedup 1.0000x reference)
import functools

import jax
import jax.numpy as jnp
from jax import lax
from jax.experimental import pallas as pl
from jax.experimental.pallas import tpu as pltpu
from jax.experimental.pallas import tpu_sc as plsc

EMB_DIM = 128
BATCH = 16384
NUM_RANK = 15
NUM_SUIT = 5
NUM_COMB = NUM_RANK * NUM_SUIT

NC = 2
NS = 16
L = 16
NW = NC * NS
BPW = BATCH // NW
CHUNK = 64
NCHUNK = BPW // CHUNK


def _prep_body(rank_ref, suit_ref, rid_ref, sid_ref, comb_ref, idx_ref):
    for r in range(NUM_RANK):
        comb_ref[pl.ds(r * NUM_SUIT, NUM_SUIT), :] = (
            suit_ref[...] + rank_ref[r, :][None, :])
    idx_ref[...] = rid_ref[...] * NUM_SUIT + sid_ref[...]


_prep = pl.pallas_call(
    _prep_body,
    out_shape=(
        jax.ShapeDtypeStruct((NUM_COMB, EMB_DIM), jnp.float32),
        jax.ShapeDtypeStruct((BATCH // EMB_DIM, EMB_DIM), jnp.int32),
    ),
)


@functools.partial(
    pl.kernel,
    mesh=plsc.VectorSubcoreMesh(core_axis_name="c", subcore_axis_name="s"),
    out_type=jax.ShapeDtypeStruct((BATCH, EMB_DIM), jnp.float32),
    scratch_types=[
        pltpu.VMEM((BPW,), jnp.int32),
        pltpu.VMEM((NCHUNK, CHUNK, EMB_DIM), jnp.float32),
        pltpu.VMEM_SHARED((NUM_COMB, EMB_DIM), jnp.float32),
        pltpu.SemaphoreType.DMA((NCHUNK,)),
        pltpu.SemaphoreType.DMA,
        pltpu.SemaphoreType.DMA,
    ],
)
def _sc_lookup(table_hbm, idx_hbm, out_hbm,
               idx_v, rows_v, table_sp, gsems, isem, osem):
    sid = lax.axis_index("s")
    wid = sid * NC + lax.axis_index("c")
    base = wid * BPW
    ld_i = pltpu.async_copy(idx_hbm.at[pl.ds(base, BPW)], idx_v, isem)

    QSIZE = 24
    for q in range(4):
        nrows = min(QSIZE, NUM_COMB - q * QSIZE)

        @pl.when(sid == q)
        def _(q=q, nrows=nrows):
            pltpu.sync_copy(table_hbm.at[pl.ds(q * QSIZE, nrows)],
                            table_sp.at[pl.ds(q * QSIZE, nrows)])

    ld_i.wait()
    plsc.subcore_barrier()
    gathers = [
        pltpu.async_copy(table_sp.at[idx_v.at[pl.ds(j * CHUNK, CHUNK)]],
                         rows_v.at[j], gsems.at[j])
        for j in range(NCHUNK)
    ]
    scatters = []
    for j in range(NCHUNK):
        gathers[j].wait()
        scatters.append(pltpu.async_copy(
            rows_v.at[j], out_hbm.at[pl.ds(base + j * CHUNK, CHUNK)], osem))
    for s in scatters:
        s.wait()


def kernel(rank_id, suit_id, rank_emb, suit_emb):
    rid = rank_id.astype(jnp.int32).reshape(BATCH // EMB_DIM, EMB_DIM)
    sid = suit_id.astype(jnp.int32).reshape(BATCH // EMB_DIM, EMB_DIM)
    comb, idx = _prep(rank_emb, suit_emb, rid, sid)
    return _sc_lookup(comb, idx.reshape(BATCH))
